# Initial kernel scaffold; baseline (speedup 1.0000x reference)
import jax
import jax.numpy as jnp
from jax.experimental import pallas as pl


def kernel(x, edge_index, W0, W1, W2, b0, b1, b2, alphas):
    raise NotImplementedError("write your pallas kernel here")


# speedup vs baseline: 8.4501x; 8.4501x over previous
import functools

import jax
import jax.numpy as jnp
from jax import lax
from jax.experimental import pallas as pl
from jax.experimental.pallas import tpu as pltpu
from jax.experimental.pallas import tpu_sc as plsc

_NC = 2
_NS = 16
_NW = _NC * _NS
_CHUNK = 128


@functools.lru_cache(maxsize=None)
def _deg_kernel(n_pad: int, k: int):
    rpt = n_pad // _NS
    mesh = plsc.VectorSubcoreMesh(core_axis_name="c", subcore_axis_name="s")

    def body(dsts, degp, dacc, dst_v, fill_v):
        c = lax.axis_index("c")
        s = lax.axis_index("s")
        w = s * _NC + c

        def fill(i, _):
            fill_v[pl.ds(i * 16, 16)] = jnp.ones((16,), jnp.float32)
            return 0

        lax.fori_loop(0, fill_v.shape[0] // 16, fill, 0)
        pltpu.sync_copy(fill_v.at[pl.ds(0, rpt)], dacc.at[pl.ds(s * rpt, rpt)])
        pltpu.sync_copy(dsts.at[w], dst_v)
        plsc.subcore_barrier()

        def step(j, _):
            pltpu.sync_copy(fill_v.at[pl.ds(0, _CHUNK)], dacc.at[dst_v.at[j]],
                            add=True)
            return 0

        lax.fori_loop(0, k, step, 0)
        plsc.subcore_barrier()
        pltpu.sync_copy(dacc.at[pl.ds(s * rpt, rpt)],
                        degp.at[pl.ds(c * n_pad + s * rpt, rpt)])

    fill_len = ((max(rpt, _CHUNK) + 15) // 16) * 16
    return pl.kernel(
        body,
        out_type=jax.ShapeDtypeStruct((_NC * n_pad,), jnp.float32),
        mesh=mesh,
        scratch_types=[
            pltpu.VMEM_SHARED((n_pad,), jnp.float32),
            pltpu.VMEM((k, _CHUNK), jnp.int32),
            pltpu.VMEM((fill_len,), jnp.float32),
        ],
    )


@functools.lru_cache(maxsize=None)
def _edge_kernel(n_pad: int, d: int, k: int):
    rpt = n_pad // _NS
    mesh = plsc.VectorSubcoreMesh(core_axis_name="c", subcore_axis_name="s")

    def body(t_hbm, srcs, dsts, zp, acc, src_v, dst_v, rows):
        c = lax.axis_index("c")
        s = lax.axis_index("s")
        w = s * _NC + c
        pltpu.sync_copy(t_hbm.at[pl.ds(s * rpt, rpt)],
                        acc.at[pl.ds(s * rpt, rpt)])
        pltpu.sync_copy(srcs.at[w], src_v)
        pltpu.sync_copy(dsts.at[w], dst_v)
        plsc.subcore_barrier()

        def step(j, _):
            pltpu.sync_copy(t_hbm.at[src_v.at[j]], rows)
            pltpu.sync_copy(rows, acc.at[dst_v.at[j]], add=True)
            return 0

        lax.fori_loop(0, k, step, 0)
        plsc.subcore_barrier()
        pltpu.sync_copy(acc.at[pl.ds(s * rpt, rpt)],
                        zp.at[c, pl.ds(s * rpt, rpt)])

    return pl.kernel(
        body,
        out_type=jax.ShapeDtypeStruct((_NC, n_pad, d), jnp.float32),
        mesh=mesh,
        scratch_types=[
            pltpu.VMEM_SHARED((n_pad, d), jnp.float32),
            pltpu.VMEM((k, _CHUNK), jnp.int32),
            pltpu.VMEM((k, _CHUNK), jnp.int32),
            pltpu.VMEM((_CHUNK, d), jnp.float32),
        ],
    )


_RB = 128


def _tc_first_body(x_ref, w_ref, degb_ref, alpha_ref, t_ref, res_ref, dinv_ref):
    deg = degb_ref[0] + degb_ref[1] - 1.0
    dinv = lax.rsqrt(deg)
    dinv_ref[...] = dinv
    xb = x_ref[...]
    res_ref[...] = alpha_ref[0, 0] * xb
    t_ref[...] = dinv * jnp.dot(xb, w_ref[...],
                                preferred_element_type=jnp.float32)


def _tc_mid_body(zp_ref, t_ref, dinv_ref, res_ref, w_ref, b_ref, alpha_ref,
                 tn_ref, resn_ref):
    dinv = dinv_ref[...]
    h = dinv * (zp_ref[0] + zp_ref[1] - t_ref[...]) + b_ref[...]
    resn_ref[...] = res_ref[...] + alpha_ref[0, 0] * h
    tn_ref[...] = dinv * jnp.dot(h, w_ref[...],
                                 preferred_element_type=jnp.float32)


def _tc_last_body(zp_ref, t_ref, dinv_ref, res_ref, b_ref, alpha_ref,
                  resn_ref):
    h = dinv_ref[...] * (zp_ref[0] + zp_ref[1] - t_ref[...]) + b_ref[...]
    resn_ref[...] = res_ref[...] + alpha_ref[0, 0] * h


def _row_spec(d):
    return pl.BlockSpec((_RB, d), lambda i: (i, 0))


def _pair_spec(d):
    return pl.BlockSpec((2, _RB, d), lambda i: (0, i, 0))


def _full_spec(shape):
    return pl.BlockSpec(shape, lambda i: tuple(0 for _ in shape))


def _smem_spec():
    return pl.BlockSpec(memory_space=pltpu.MemorySpace.SMEM)


@functools.lru_cache(maxsize=None)
def _tc_first(n_pad: int, d: int):
    g = n_pad // _RB
    return pl.pallas_call(
        _tc_first_body,
        grid=(g,),
        in_specs=[_row_spec(d), _full_spec((d, d)), _pair_spec(d),
                  _smem_spec()],
        out_specs=[_row_spec(d), _row_spec(d), _row_spec(d)],
        out_shape=[jax.ShapeDtypeStruct((n_pad, d), jnp.float32)] * 3,
    )


@functools.lru_cache(maxsize=None)
def _tc_mid(n_pad: int, d: int):
    g = n_pad // _RB
    return pl.pallas_call(
        _tc_mid_body,
        grid=(g,),
        in_specs=[_pair_spec(d), _row_spec(d), _row_spec(d), _row_spec(d),
                  _full_spec((d, d)), _full_spec((1, d)), _smem_spec()],
        out_specs=[_row_spec(d), _row_spec(d)],
        out_shape=[jax.ShapeDtypeStruct((n_pad, d), jnp.float32)] * 2,
    )


@functools.lru_cache(maxsize=None)
def _tc_last(n_pad: int, d: int):
    g = n_pad // _RB
    return pl.pallas_call(
        _tc_last_body,
        grid=(g,),
        in_specs=[_pair_spec(d), _row_spec(d), _row_spec(d), _row_spec(d),
                  _full_spec((1, d)), _smem_spec()],
        out_specs=_row_spec(d),
        out_shape=jax.ShapeDtypeStruct((n_pad, d), jnp.float32),
    )


def kernel(x, edge_index, W0, W1, W2, b0, b1, b2, alphas):
    n, d = x.shape
    src, dst = edge_index[0], edge_index[1]
    e = src.shape[0]

    ept = -(-e // _NW)
    k = -(-ept // _CHUNK)
    e_pad = _NW * k * _CHUNK
    pad = e_pad - e
    src_p = jnp.concatenate(
        [src, jnp.zeros((pad,), jnp.int32)]).reshape(_NW, k, _CHUNK)
    dst_p = jnp.concatenate(
        [dst, jnp.full((pad,), n, jnp.int32)]).reshape(_NW, k, _CHUNK)

    n_pad = -(-(n + 1) // 2048) * 2048
    x_pad = jnp.pad(x, ((0, n_pad - n), (0, 0)))

    degp = _deg_kernel(n_pad, k)(dst_p).reshape(2, n_pad)
    degb = jnp.broadcast_to(degp[:, :, None], (2, n_pad, d))

    a = [alphas[i].reshape(1, 1) for i in range(4)]
    t0, res, dinv = _tc_first(n_pad, d)(x_pad, W0, degb, a[0])
    zp = _edge_kernel(n_pad, d, k)(t0, src_p, dst_p)
    t1, res = _tc_mid(n_pad, d)(zp, t0, dinv, res, W1,
                                b0.reshape(1, d), a[1])
    zp = _edge_kernel(n_pad, d, k)(t1, src_p, dst_p)
    t2, res = _tc_mid(n_pad, d)(zp, t1, dinv, res, W2,
                                b1.reshape(1, d), a[2])
    zp = _edge_kernel(n_pad, d, k)(t2, src_p, dst_p)
    res = _tc_last(n_pad, d)(zp, t2, dinv, res, b2.reshape(1, d), a[3])
    return res[:n]
